# baseline (speedup 1.0000x reference)
import functools

import jax
import jax.numpy as jnp
from jax import lax
from jax.experimental import pallas as pl
from jax.experimental.pallas import tpu as pltpu
from jax.experimental.pallas import tpu_sc as plsc

_B = 128
_N = 32768
_K_CUT = 3276
_NCHUNK = _N // 16
_MININT = -(2**31)
_NW = 32
_ROWS_PER_W = _B // _NW


def _sc_dropmax_body(x_hbm, o_hbm, row_v, key_v, hist_v):
    wid = lax.axis_index("s") * 2 + lax.axis_index("c")
    lane = lax.iota(jnp.int32, 16)
    lane_base = lane * 2048
    ones = jnp.ones((16,), jnp.int32)
    zeros = jnp.zeros((16,), jnp.int32)

    def clear_hist(n_bins):
        def clr(i, c):
            hist_v[pl.ds(i * 16, 16)] = zeros
            return c

        lax.fori_loop(0, n_bins, clr, 0)

    def scan_level(n_chunks, k_want, lane_stride=2048):
        def body(jj, carry):
            s, dstar, above = carry
            j = n_chunks - 1 - jj
            tot = hist_v[pl.ds(16 * j, 16)]
            for l in range(1, 16):
                tot = tot + hist_v[pl.ds(l * lane_stride + 16 * j, 16)]
            sfx = lax.rev(jnp.cumsum(lax.rev(tot, (0,)), axis=0), (0,))
            gi = s + sfx
            ge = gi - tot
            cond = (gi >= k_want) & (ge < k_want)
            dloc = jnp.where(cond, j * 16 + lane, -1)
            dstar = jnp.maximum(dstar, jnp.max(dloc))
            above = jnp.maximum(above, jnp.max(jnp.where(cond, ge, -1)))
            s = s + jnp.sum(tot)
            return s, dstar, above

        _, dstar, above = lax.fori_loop(
            0, n_chunks, body, (jnp.int32(0), jnp.int32(-1), jnp.int32(-1))
        )
        return dstar, above

    for i in range(_ROWS_PER_W):
        r = wid * _ROWS_PER_W + i
        base = r * _N
        pltpu.sync_copy(x_hbm.at[pl.ds(base, _N)], row_v)

        clear_hist(_NCHUNK)

        def pass_a(ci, c):
            f = row_v[pl.ds(ci * 16, 16)]
            b = lax.bitcast_convert_type(f, jnp.int32)
            uk = jnp.where(b < 0, ~b, b ^ _MININT)
            key_v[pl.ds(ci * 16, 16)] = uk
            d1 = lax.shift_right_logical(uk, 21)
            plsc.addupdate_scatter(hist_v, [lane_base + d1], ones)
            return c

        lax.fori_loop(0, _NCHUNK, pass_a, 0)
        d1s, above1 = scan_level(128, jnp.int32(_K_CUT))
        k2 = _K_CUT - above1

        clear_hist(_NCHUNK)

        def pass_b(ci, c):
            uk = key_v[pl.ds(ci * 16, 16)]
            m = lax.shift_right_logical(uk, 21) == d1s
            d2 = lax.shift_right_logical(uk, 10) & 0x7FF
            plsc.addupdate_scatter(hist_v, [lane_base + d2], ones, mask=m)
            return c

        lax.fori_loop(0, _NCHUNK, pass_b, 0)
        d2s, above2 = scan_level(128, k2)
        k3 = k2 - above2
        prefix22 = d1s * 2048 + d2s

        clear_hist(1024)

        def pass_c(ci, c):
            uk = key_v[pl.ds(ci * 16, 16)]
            m = lax.shift_right_logical(uk, 10) == prefix22
            d3 = uk & 0x3FF
            plsc.addupdate_scatter(hist_v, [lane * 1024 + d3], ones, mask=m)
            return c

        lax.fori_loop(0, _NCHUNK, pass_c, 0)
        d3s, _ = scan_level(64, k3, lane_stride=1024)

        t_s = ((prefix22 * 1024) | d3s) ^ _MININT

        def pass_d(ci, c):
            uk = key_v[pl.ds(ci * 16, 16)]
            ks = uk ^ _MININT
            f = row_v[pl.ds(ci * 16, 16)]
            row_v[pl.ds(ci * 16, 16)] = jnp.where(ks >= t_s, jnp.float32(0.0), f)
            return c

        lax.fori_loop(0, _NCHUNK, pass_d, 0)
        pltpu.sync_copy(row_v, o_hbm.at[pl.ds(base, _N)])


@jax.jit
def _dropmax_sc(x):
    mesh = plsc.VectorSubcoreMesh(core_axis_name="c", subcore_axis_name="s")
    k = functools.partial(
        pl.kernel,
        mesh=mesh,
        out_type=jax.ShapeDtypeStruct((_B * _N,), jnp.float32),
        scratch_types=[
            pltpu.VMEM((_N,), jnp.float32),
            pltpu.VMEM((_N,), jnp.int32),
            pltpu.VMEM((_N,), jnp.int32),
        ],
        compiler_params=pltpu.CompilerParams(needs_layout_passes=False),
    )(_sc_dropmax_body)
    return k(x.reshape(-1)).reshape(_B, _N)


_ROWS_PER_BLOCK = 16


def _dropmax_block(x_ref, o_ref):
    x = x_ref[...]
    bits = jax.lax.bitcast_convert_type(x, jnp.int32)
    key = jnp.where(bits < 0, bits ^ jnp.int32(0x7FFFFFFF), bits)

    lo0 = jnp.full((x.shape[0], 1), jnp.iinfo(jnp.int32).min, jnp.int32)
    hi0 = jnp.full((x.shape[0], 1), jnp.iinfo(jnp.int32).max, jnp.int32)

    def body(_, carry):
        lo, hi = carry
        mid = (lo | hi) - ((lo ^ hi) >> 1)
        cnt = jnp.sum((key >= mid).astype(jnp.int32), axis=1, keepdims=True)
        pred = cnt >= _K_CUT
        lo = jnp.where(pred, mid, lo)
        hi = jnp.where(pred, hi, mid - 1)
        return lo, hi

    lo, _ = jax.lax.fori_loop(0, 32, body, (lo0, hi0))
    o_ref[...] = jnp.where(key >= lo, jnp.float32(0.0), x)


@jax.jit
def _dropmax_tc(x):
    b, n = x.shape
    grid = b // _ROWS_PER_BLOCK
    return pl.pallas_call(
        _dropmax_block,
        grid=(grid,),
        in_specs=[pl.BlockSpec((_ROWS_PER_BLOCK, n), lambda i: (i, 0))],
        out_specs=pl.BlockSpec((_ROWS_PER_BLOCK, n), lambda i: (i, 0)),
        out_shape=jax.ShapeDtypeStruct((b, n), jnp.float32),
    )(x)


def kernel(x):
    return _dropmax_sc(x)


# speedup vs baseline: 3.2887x; 3.2887x over previous
import functools

import jax
import jax.numpy as jnp
from jax import lax
from jax.experimental import pallas as pl
from jax.experimental.pallas import tpu as pltpu
from jax.experimental.pallas import tpu_sc as plsc

_B = 128
_N = 32768
_K_CUT = 3276
_NCHUNK = _N // 16
_MININT = -(2**31)
_NW = 32
_ROWS_PER_W = _B // _NW


def _sc_dropmax_body(rows_per_w, x_hbm, o_hbm, row_v, key_v, hist_v):
    wid = lax.axis_index("s") * 2 + lax.axis_index("c")
    lane = lax.iota(jnp.int32, 16)
    lane_base = lane * 2048
    ones = jnp.ones((16,), jnp.int32)
    zeros = jnp.zeros((16,), jnp.int32)

    def clear_hist(n_bins):
        def clr(i, c):
            hist_v[pl.ds(i * 16, 16)] = zeros
            return c

        lax.fori_loop(0, n_bins, clr, 0)

    def scan_level(n_chunks, k_want, lane_stride=2048):
        def body(jj, carry):
            s, dstar, above = carry
            j = n_chunks - 1 - jj
            tot = hist_v[pl.ds(16 * j, 16)]
            for l in range(1, 16):
                tot = tot + hist_v[pl.ds(l * lane_stride + 16 * j, 16)]
            sfx = lax.rev(jnp.cumsum(lax.rev(tot, (0,)), axis=0), (0,))
            gi = s + sfx
            ge = gi - tot
            cond = (gi >= k_want) & (ge < k_want)
            dloc = jnp.where(cond, j * 16 + lane, -1)
            dstar = jnp.maximum(dstar, jnp.max(dloc))
            above = jnp.maximum(above, jnp.max(jnp.where(cond, ge, -1)))
            s = s + jnp.sum(tot)
            return s, dstar, above

        _, dstar, above = lax.fori_loop(
            0, n_chunks, body, (jnp.int32(0), jnp.int32(-1), jnp.int32(-1))
        )
        return dstar, above

    for i in range(rows_per_w):
        r = wid * rows_per_w + i
        base = r * _N
        pltpu.sync_copy(x_hbm.at[pl.ds(base, _N)], row_v)

        clear_hist(_NCHUNK)

        def pass_a(ci, c):
            f = row_v[pl.ds(ci * 16, 16)]
            b = lax.bitcast_convert_type(f, jnp.int32)
            uk = jnp.where(b < 0, ~b, b ^ _MININT)
            key_v[pl.ds(ci * 16, 16)] = uk
            d1 = lax.shift_right_logical(uk, 21)
            plsc.addupdate_scatter(hist_v, [lane_base + d1], ones)
            return c

        lax.fori_loop(0, _NCHUNK, pass_a, 0)
        d1s, above1 = scan_level(128, jnp.int32(_K_CUT))
        k2 = _K_CUT - above1

        clear_hist(_NCHUNK)

        def pass_b(ci, c):
            uk = key_v[pl.ds(ci * 16, 16)]
            m = lax.shift_right_logical(uk, 21) == d1s
            d2 = lax.shift_right_logical(uk, 10) & 0x7FF
            plsc.addupdate_scatter(hist_v, [lane_base + d2], ones, mask=m)
            return c

        lax.fori_loop(0, _NCHUNK, pass_b, 0)
        d2s, above2 = scan_level(128, k2)
        k3 = k2 - above2
        prefix22 = d1s * 2048 + d2s

        clear_hist(1024)

        def pass_c(ci, c):
            uk = key_v[pl.ds(ci * 16, 16)]
            m = lax.shift_right_logical(uk, 10) == prefix22
            d3 = uk & 0x3FF
            plsc.addupdate_scatter(hist_v, [lane * 1024 + d3], ones, mask=m)
            return c

        lax.fori_loop(0, _NCHUNK, pass_c, 0)
        d3s, _ = scan_level(64, k3, lane_stride=1024)

        t_s = ((prefix22 * 1024) | d3s) ^ _MININT

        def pass_d(ci, c):
            uk = key_v[pl.ds(ci * 16, 16)]
            ks = uk ^ _MININT
            f = row_v[pl.ds(ci * 16, 16)]
            row_v[pl.ds(ci * 16, 16)] = jnp.where(ks >= t_s, jnp.float32(0.0), f)
            return c

        lax.fori_loop(0, _NCHUNK, pass_d, 0)
        pltpu.sync_copy(row_v, o_hbm.at[pl.ds(base, _N)])


def _dropmax_sc_rows(x_rows):
    n_rows = x_rows.shape[0]
    mesh = plsc.VectorSubcoreMesh(core_axis_name="c", subcore_axis_name="s")
    k = functools.partial(
        pl.kernel,
        mesh=mesh,
        out_type=jax.ShapeDtypeStruct((n_rows * _N,), jnp.float32),
        scratch_types=[
            pltpu.VMEM((_N,), jnp.float32),
            pltpu.VMEM((_N,), jnp.int32),
            pltpu.VMEM((_N,), jnp.int32),
        ],
        compiler_params=pltpu.CompilerParams(needs_layout_passes=False),
    )(functools.partial(_sc_dropmax_body, n_rows // _NW))
    return k(x_rows.reshape(-1)).reshape(n_rows, _N)


@jax.jit
def _dropmax_sc(x):
    return _dropmax_sc_rows(x)


_ROWS_PER_BLOCK = 16


def _dropmax_block(x_ref, o_ref):
    x = x_ref[...]
    bits = jax.lax.bitcast_convert_type(x, jnp.int32)
    key = jnp.where(bits < 0, bits ^ jnp.int32(0x7FFFFFFF), bits)

    lo0 = jnp.full((x.shape[0], 1), jnp.iinfo(jnp.int32).min, jnp.int32)
    hi0 = jnp.full((x.shape[0], 1), jnp.iinfo(jnp.int32).max, jnp.int32)

    def body(_, carry):
        lo, hi = carry
        mid = (lo | hi) - ((lo ^ hi) >> 1)
        cnt = jnp.sum((key >= mid).astype(jnp.int32), axis=1, keepdims=True)
        pred = cnt >= _K_CUT
        lo = jnp.where(pred, mid, lo)
        hi = jnp.where(pred, hi, mid - 1)
        return lo, hi

    lo, _ = jax.lax.fori_loop(0, 32, body, (lo0, hi0))
    o_ref[...] = jnp.where(key >= lo, jnp.float32(0.0), x)


@jax.jit
def _dropmax_tc(x):
    b, n = x.shape
    grid = b // _ROWS_PER_BLOCK
    return pl.pallas_call(
        _dropmax_block,
        grid=(grid,),
        in_specs=[pl.BlockSpec((_ROWS_PER_BLOCK, n), lambda i: (i, 0))],
        out_specs=pl.BlockSpec((_ROWS_PER_BLOCK, n), lambda i: (i, 0)),
        out_shape=jax.ShapeDtypeStruct((b, n), jnp.float32),
    )(x)


_SC_ROWS = 32


@jax.jit
def _dropmax_hybrid(x):
    sc_out = _dropmax_sc_rows(x[:_SC_ROWS])
    tc_out = _dropmax_tc(x[_SC_ROWS:])
    return jnp.concatenate([sc_out, tc_out], axis=0)


def kernel(x):
    return _dropmax_hybrid(x)


# speedup vs baseline: 3.5901x; 1.0916x over previous
import functools

import jax
import jax.numpy as jnp
from jax import lax
from jax.experimental import pallas as pl
from jax.experimental.pallas import tpu as pltpu
from jax.experimental.pallas import tpu_sc as plsc

_B = 128
_N = 32768
_K_CUT = 3276
_NCHUNK = _N // 16
_MININT = -(2**31)
_NW = 32
_ROWS_PER_W = _B // _NW


def _sc_dropmax_body(rows_per_w, x_hbm, o_hbm, row_v, key_v, hist_v, tot_v):
    wid = lax.axis_index("s") * 2 + lax.axis_index("c")
    lane = lax.iota(jnp.int32, 16)
    lane2048 = lane * 2048
    lane1024 = lane * 1024
    ones = jnp.ones((16,), jnp.int32)
    zeros = jnp.zeros((16,), jnp.int32)

    @plsc.parallel_loop(0, _NCHUNK, unroll=8)
    def _(ci):
        hist_v[pl.ds(ci * 16, 16)] = zeros

    def scan_level(n_chunks, k_want, lane_stride):
        @plsc.parallel_loop(
            0, n_chunks, carry=(jnp.int32(0), jnp.int32(0), jnp.int32(0))
        )
        def p1(jj, c):
            s, jstar, above_c = c
            j = n_chunks - 1 - jj
            tot = hist_v[pl.ds(16 * j, 16)]
            hist_v[pl.ds(16 * j, 16)] = zeros
            for l in range(1, 16):
                off = l * lane_stride + 16 * j
                v = hist_v[pl.ds(off, 16)]
                hist_v[pl.ds(off, 16)] = zeros
                tot = tot + v
            tot_v[pl.ds(16 * j, 16)] = tot
            s2 = s + jnp.sum(tot)
            hit = (s2 >= k_want) & (s < k_want)
            jstar = jnp.where(hit, j, jstar)
            above_c = jnp.where(hit, s, above_c)
            return s2, jstar, above_c

        _, jstar, above_c = p1
        tot = tot_v[pl.ds(16 * jstar, 16)]
        sfx = lax.rev(jnp.cumsum(lax.rev(tot, (0,)), axis=0), (0,))
        gi = above_c + sfx
        ge = gi - tot
        cond = (gi >= k_want) & (ge < k_want)
        dstar = jstar * 16 + jnp.max(jnp.where(cond, lane, -1))
        above = jnp.max(jnp.where(cond, ge, -1))
        return dstar, above

    for i in range(rows_per_w):
        r = wid * rows_per_w + i
        base = r * _N
        pltpu.sync_copy(x_hbm.at[pl.ds(base, _N)], row_v)

        @plsc.parallel_loop(0, _NCHUNK, unroll=8)
        def _(ci):
            f = row_v[pl.ds(ci * 16, 16)]
            b = lax.bitcast_convert_type(f, jnp.int32)
            uk = jnp.where(b < 0, ~b, b ^ _MININT)
            key_v[pl.ds(ci * 16, 16)] = uk
            d1 = lax.shift_right_logical(uk, 21)
            plsc.addupdate_scatter(hist_v, [lane2048 + d1], ones)

        d1s, above1 = scan_level(128, jnp.int32(_K_CUT), 2048)
        k2 = _K_CUT - above1

        @plsc.parallel_loop(0, _NCHUNK, unroll=8)
        def _(ci):
            uk = key_v[pl.ds(ci * 16, 16)]
            m = lax.shift_right_logical(uk, 21) == d1s
            d2 = lax.shift_right_logical(uk, 10) & 0x7FF
            plsc.addupdate_scatter(hist_v, [lane2048 + d2], ones, mask=m)

        d2s, above2 = scan_level(128, k2, 2048)
        k3 = k2 - above2
        prefix22 = d1s * 2048 + d2s

        @plsc.parallel_loop(0, _NCHUNK, unroll=8)
        def _(ci):
            uk = key_v[pl.ds(ci * 16, 16)]
            m = lax.shift_right_logical(uk, 10) == prefix22
            d3 = uk & 0x3FF
            plsc.addupdate_scatter(hist_v, [lane1024 + d3], ones, mask=m)

        d3s, _ = scan_level(64, k3, 1024)

        t_s = ((prefix22 * 1024) | d3s) ^ _MININT

        @plsc.parallel_loop(0, _NCHUNK, unroll=8)
        def _(ci):
            uk = key_v[pl.ds(ci * 16, 16)]
            ks = uk ^ _MININT
            f = row_v[pl.ds(ci * 16, 16)]
            row_v[pl.ds(ci * 16, 16)] = jnp.where(ks >= t_s, jnp.float32(0.0), f)

        pltpu.sync_copy(row_v, o_hbm.at[pl.ds(base, _N)])


def _dropmax_sc_rows(x_rows):
    n_rows = x_rows.shape[0]
    mesh = plsc.VectorSubcoreMesh(core_axis_name="c", subcore_axis_name="s")
    k = functools.partial(
        pl.kernel,
        mesh=mesh,
        out_type=jax.ShapeDtypeStruct((n_rows * _N,), jnp.float32),
        scratch_types=[
            pltpu.VMEM((_N,), jnp.float32),
            pltpu.VMEM((_N,), jnp.int32),
            pltpu.VMEM((_N,), jnp.int32),
            pltpu.VMEM((_NCHUNK,), jnp.int32),
        ],
        compiler_params=pltpu.CompilerParams(needs_layout_passes=False),
    )(functools.partial(_sc_dropmax_body, n_rows // _NW))
    return k(x_rows.reshape(-1)).reshape(n_rows, _N)


@jax.jit
def _dropmax_sc(x):
    return _dropmax_sc_rows(x)


_ROWS_PER_BLOCK = 16


def _dropmax_block(x_ref, o_ref):
    x = x_ref[...]
    bits = jax.lax.bitcast_convert_type(x, jnp.int32)
    key = jnp.where(bits < 0, bits ^ jnp.int32(0x7FFFFFFF), bits)

    lo0 = jnp.full((x.shape[0], 1), jnp.iinfo(jnp.int32).min, jnp.int32)
    hi0 = jnp.full((x.shape[0], 1), jnp.iinfo(jnp.int32).max, jnp.int32)

    def body(_, carry):
        lo, hi = carry
        mid = (lo | hi) - ((lo ^ hi) >> 1)
        cnt = jnp.sum((key >= mid).astype(jnp.int32), axis=1, keepdims=True)
        pred = cnt >= _K_CUT
        lo = jnp.where(pred, mid, lo)
        hi = jnp.where(pred, hi, mid - 1)
        return lo, hi

    lo, _ = jax.lax.fori_loop(0, 32, body, (lo0, hi0))
    o_ref[...] = jnp.where(key >= lo, jnp.float32(0.0), x)


@jax.jit
def _dropmax_tc(x):
    b, n = x.shape
    grid = b // _ROWS_PER_BLOCK
    return pl.pallas_call(
        _dropmax_block,
        grid=(grid,),
        in_specs=[pl.BlockSpec((_ROWS_PER_BLOCK, n), lambda i: (i, 0))],
        out_specs=pl.BlockSpec((_ROWS_PER_BLOCK, n), lambda i: (i, 0)),
        out_shape=jax.ShapeDtypeStruct((b, n), jnp.float32),
    )(x)


_SC_ROWS = 32


@jax.jit
def _dropmax_hybrid(x):
    sc_out = _dropmax_sc_rows(x[:_SC_ROWS])
    tc_out = _dropmax_tc(x[_SC_ROWS:])
    return jnp.concatenate([sc_out, tc_out], axis=0)


def kernel(x):
    return _dropmax_hybrid(x)


# speedup vs baseline: 4.3030x; 1.1986x over previous
import functools

import jax
import jax.numpy as jnp
from jax import lax
from jax.experimental import pallas as pl
from jax.experimental.pallas import tpu as pltpu
from jax.experimental.pallas import tpu_sc as plsc

_B = 128
_N = 32768
_K_CUT = 3276
_NCHUNK = _N // 16
_MININT = -(2**31)
_NW = 32
_ROWS_PER_W = _B // _NW


def _sc_dropmax_body(rows_per_w, x_hbm, o_hbm, row_v, key_v, hist_v, tot_v):
    wid = lax.axis_index("s") * 2 + lax.axis_index("c")
    lane = lax.iota(jnp.int32, 16)
    lane2048 = lane * 2048
    lane1024 = lane * 1024
    ones = jnp.ones((16,), jnp.int32)
    zeros = jnp.zeros((16,), jnp.int32)

    @plsc.parallel_loop(0, _NCHUNK, unroll=8)
    def _(ci):
        hist_v[pl.ds(ci * 16, 16)] = zeros

    def scan_level(n_chunks, k_want, lane_stride):
        @plsc.parallel_loop(
            0, n_chunks, carry=(jnp.int32(0), jnp.int32(0), jnp.int32(0))
        )
        def p1(jj, c):
            s, jstar, above_c = c
            j = n_chunks - 1 - jj
            tot = hist_v[pl.ds(16 * j, 16)]
            hist_v[pl.ds(16 * j, 16)] = zeros
            for l in range(1, 16):
                off = l * lane_stride + 16 * j
                v = hist_v[pl.ds(off, 16)]
                hist_v[pl.ds(off, 16)] = zeros
                tot = tot + v
            tot_v[pl.ds(16 * j, 16)] = tot
            s2 = s + jnp.sum(tot)
            hit = (s2 >= k_want) & (s < k_want)
            jstar = jnp.where(hit, j, jstar)
            above_c = jnp.where(hit, s, above_c)
            return s2, jstar, above_c

        _, jstar, above_c = p1
        tot = tot_v[pl.ds(16 * jstar, 16)]
        sfx = lax.rev(jnp.cumsum(lax.rev(tot, (0,)), axis=0), (0,))
        gi = above_c + sfx
        ge = gi - tot
        cond = (gi >= k_want) & (ge < k_want)
        dstar = jstar * 16 + jnp.max(jnp.where(cond, lane, -1))
        above = jnp.max(jnp.where(cond, ge, -1))
        return dstar, above

    for i in range(rows_per_w):
        r = wid * rows_per_w + i
        base = r * _N
        pltpu.sync_copy(x_hbm.at[pl.ds(base, _N)], row_v)

        @plsc.parallel_loop(0, _NCHUNK, unroll=8)
        def _(ci):
            f = row_v[pl.ds(ci * 16, 16)]
            b = lax.bitcast_convert_type(f, jnp.int32)
            uk = jnp.where(b < 0, ~b, b ^ _MININT)
            key_v[pl.ds(ci * 16, 16)] = uk
            d1 = lax.shift_right_logical(uk, 21)
            plsc.addupdate_scatter(hist_v, [lane2048 + d1], ones)

        d1s, above1 = scan_level(128, jnp.int32(_K_CUT), 2048)
        k2 = _K_CUT - above1

        @plsc.parallel_loop(0, _NCHUNK, unroll=8)
        def _(ci):
            uk = key_v[pl.ds(ci * 16, 16)]
            m = lax.shift_right_logical(uk, 21) == d1s
            d2 = lax.shift_right_logical(uk, 10) & 0x7FF
            plsc.addupdate_scatter(hist_v, [lane2048 + d2], ones, mask=m)

        d2s, above2 = scan_level(128, k2, 2048)
        k3 = k2 - above2
        prefix22 = d1s * 2048 + d2s

        @plsc.parallel_loop(0, _NCHUNK, unroll=8)
        def _(ci):
            uk = key_v[pl.ds(ci * 16, 16)]
            m = lax.shift_right_logical(uk, 10) == prefix22
            d3 = uk & 0x3FF
            plsc.addupdate_scatter(hist_v, [lane1024 + d3], ones, mask=m)

        d3s, _ = scan_level(64, k3, 1024)

        t_s = ((prefix22 * 1024) | d3s) ^ _MININT

        @plsc.parallel_loop(0, _NCHUNK, unroll=8)
        def _(ci):
            uk = key_v[pl.ds(ci * 16, 16)]
            ks = uk ^ _MININT
            f = row_v[pl.ds(ci * 16, 16)]
            row_v[pl.ds(ci * 16, 16)] = jnp.where(ks >= t_s, jnp.float32(0.0), f)

        pltpu.sync_copy(row_v, o_hbm.at[pl.ds(base, _N)])


def _dropmax_sc_rows(x_flat, n_rows):
    mesh = plsc.VectorSubcoreMesh(core_axis_name="c", subcore_axis_name="s")
    k = functools.partial(
        pl.kernel,
        mesh=mesh,
        out_type=jax.ShapeDtypeStruct((n_rows * _N,), jnp.float32),
        scratch_types=[
            pltpu.VMEM((_N,), jnp.float32),
            pltpu.VMEM((_N,), jnp.int32),
            pltpu.VMEM((_N,), jnp.int32),
            pltpu.VMEM((_NCHUNK,), jnp.int32),
        ],
        compiler_params=pltpu.CompilerParams(needs_layout_passes=False),
    )(functools.partial(_sc_dropmax_body, n_rows // _NW))
    return k(x_flat).reshape(n_rows, _N)


@jax.jit
def _dropmax_sc(x):
    return _dropmax_sc_rows(x.reshape(-1), _B)


_ROWS_PER_BLOCK = 16


def _dropmax_block(x_ref, o_ref):
    x = x_ref[...]
    bits = jax.lax.bitcast_convert_type(x, jnp.int32)
    key = jnp.where(bits < 0, bits ^ jnp.int32(0x7FFFFFFF), bits)

    lo0 = jnp.full((x.shape[0], 1), jnp.iinfo(jnp.int32).min, jnp.int32)
    hi0 = jnp.full((x.shape[0], 1), jnp.iinfo(jnp.int32).max, jnp.int32)

    def body(_, carry):
        lo, hi = carry
        mid = (lo | hi) - ((lo ^ hi) >> 1)
        cnt = jnp.sum((key >= mid).astype(jnp.int32), axis=1, keepdims=True)
        pred = cnt >= _K_CUT
        lo = jnp.where(pred, mid, lo)
        hi = jnp.where(pred, hi, mid - 1)
        return lo, hi

    lo, _ = jax.lax.fori_loop(0, 32, body, (lo0, hi0))
    o_ref[...] = jnp.where(key >= lo, jnp.float32(0.0), x)


@jax.jit
def _dropmax_tc(x):
    b, n = x.shape
    grid = b // _ROWS_PER_BLOCK
    return pl.pallas_call(
        _dropmax_block,
        grid=(grid,),
        in_specs=[pl.BlockSpec((_ROWS_PER_BLOCK, n), lambda i: (i, 0))],
        out_specs=pl.BlockSpec((_ROWS_PER_BLOCK, n), lambda i: (i, 0)),
        out_shape=jax.ShapeDtypeStruct((b, n), jnp.float32),
    )(x)


_SC_ROWS = 64
_SC_BLOCKS = _SC_ROWS // _ROWS_PER_BLOCK


@jax.jit
def _dropmax_hybrid(x):
    b, n = x.shape
    sc_out = _dropmax_sc_rows(x.reshape(-1), _SC_ROWS)
    tc_full = pl.pallas_call(
        _dropmax_block,
        grid=((b - _SC_ROWS) // _ROWS_PER_BLOCK,),
        in_specs=[pl.BlockSpec((_ROWS_PER_BLOCK, n), lambda i: (i + _SC_BLOCKS, 0))],
        out_specs=pl.BlockSpec((_ROWS_PER_BLOCK, n), lambda i: (i + _SC_BLOCKS, 0)),
        out_shape=jax.ShapeDtypeStruct((b, n), jnp.float32),
    )(x)
    return lax.dynamic_update_slice(tc_full, sc_out, (0, 0))


def kernel(x):
    return _dropmax_hybrid(x)
